# baseline (speedup 1.0000x reference)
import functools

import jax
import jax.numpy as jnp
from jax import lax
from jax.experimental import pallas as pl
from jax.experimental.pallas import tpu as pltpu
from jax.experimental.pallas import tpu_sc as plsc

K = 256
ROWS_PER_BLOCK = 8


def _topk_body(w_ref, vals_ref, idx_ref):
    x = w_ref[...]
    col = lax.broadcasted_iota(jnp.int32, x.shape, 1)
    kcol = lax.broadcasted_iota(jnp.int32, (x.shape[0], K), 1)
    h = x.shape[1]
    vacc = jnp.zeros((x.shape[0], K), jnp.float32)
    iacc = jnp.zeros((x.shape[0], K), jnp.int32)

    def body(t, carry):
        x, vacc, iacc = carry
        m = jnp.max(x, axis=1, keepdims=True)
        im = jnp.min(jnp.where(x == m, col, h), axis=1, keepdims=True)
        sel = kcol == t
        vacc = jnp.where(sel, m, vacc)
        iacc = jnp.where(sel, im, iacc)
        x = jnp.where(col == im, -jnp.inf, x)
        return x, vacc, iacc

    _, vacc, iacc = lax.fori_loop(0, K, body, (x, vacc, iacc))
    vals_ref[...] = vacc
    idx_ref[...] = iacc


def _row_topk(embed_weight):
    v, h = embed_weight.shape
    grid = v // ROWS_PER_BLOCK
    return pl.pallas_call(
        _topk_body,
        grid=(grid,),
        in_specs=[pl.BlockSpec((ROWS_PER_BLOCK, h), lambda i: (i, 0))],
        out_specs=[
            pl.BlockSpec((ROWS_PER_BLOCK, K), lambda i: (i, 0)),
            pl.BlockSpec((ROWS_PER_BLOCK, K), lambda i: (i, 0)),
        ],
        out_shape=[
            jax.ShapeDtypeStruct((v, K), jnp.float32),
            jax.ShapeDtypeStruct((v, K), jnp.int32),
        ],
    )(embed_weight)


CHUNK = 128


@functools.cache
def _gather_fn(n_tokens):
    info = plsc.get_sparse_core_info()
    nc, ns = info.num_cores, info.num_subcores
    nw = nc * ns
    bpw = n_tokens // nw
    nchunks = bpw // CHUNK
    mesh = plsc.VectorSubcoreMesh(core_axis_name="c", subcore_axis_name="s")

    @functools.partial(
        pl.kernel,
        mesh=mesh,
        out_type=(
            jax.ShapeDtypeStruct((n_tokens, K), jnp.float32),
            jax.ShapeDtypeStruct((n_tokens, K), jnp.int32),
        ),
        scratch_types=[
            pltpu.VMEM((CHUNK,), jnp.int32),
            pltpu.VMEM((CHUNK, K), jnp.float32),
            pltpu.VMEM((CHUNK, K), jnp.int32),
            pltpu.SemaphoreType.DMA,
            pltpu.SemaphoreType.DMA,
        ],
    )
    def gather(tv_hbm, ti_hbm, ids_hbm, ov_hbm, oi_hbm, ids_v, vv, iv, s1, s2):
        wid = lax.axis_index("s") * nc + lax.axis_index("c")
        base = wid * bpw
        for c in range(nchunks):
            off = base + c * CHUNK
            pltpu.sync_copy(ids_hbm.at[pl.ds(off, CHUNK)], ids_v)
            cp1 = pltpu.async_copy(tv_hbm.at[ids_v], vv, s1)
            cp2 = pltpu.async_copy(ti_hbm.at[ids_v], iv, s2)
            cp1.wait()
            cp2.wait()
            pltpu.sync_copy(vv, ov_hbm.at[pl.ds(off, CHUNK)])
            pltpu.sync_copy(iv, oi_hbm.at[pl.ds(off, CHUNK)])

    return gather


def kernel(input_ids, embed_weight):
    b, s = input_ids.shape
    tvals, tidx = _row_topk(embed_weight)
    ov, oi = _gather_fn(b * s)(tvals, tidx, input_ids.reshape(-1))
    return ov.reshape(b, s, K), oi.reshape(b, s, K)


# speedup vs baseline: 5.0180x; 5.0180x over previous
import functools

import jax
import jax.numpy as jnp
from jax import lax
from jax.experimental import pallas as pl
from jax.experimental.pallas import tpu as pltpu
from jax.experimental.pallas import tpu_sc as plsc

K = 256
H = 4096
ROWS_PER_BLOCK = 8


def _stages():
    st = []
    kk = 2
    while kk <= K:
        j = kk // 2
        while j >= 1:
            st.append(("p1", j, kk))
            j //= 2
        kk *= 2
    for level in (1, 2, 3, 4):
        st.append(("cl", 256 << (level - 1), level))
        j = 128
        while j >= 1:
            st.append(("mg", j, level))
            j //= 2
    return st


_STAGES = _stages()


def _xor_shuffle(x, dist, lane):
    left = jnp.roll(x, -dist, axis=1)
    right = jnp.roll(x, dist, axis=1)
    return jnp.where((lane & dist) != 0, right, left)


def _topk_sort_block(x, perm):
    r = x.shape[0]
    lane = lax.broadcasted_iota(jnp.int32, (r, H), 1)
    s = (lane & 127) * 32 + (lane >> 7)
    key = x
    idx = lane

    for kind, d, prm in _STAGES:
        dist = 128 * d if d < 32 else d // 32
        if kind == "p1":
            winner = ((s & d) != 0) ^ ((s & prm) != 0)
        elif kind == "cl":
            winner = (s & d) == 0
        elif prm == 4:
            winner = (s & d) == 0
        else:
            winner = ((s & d) != 0) ^ (((s >> (8 + prm)) & 1) == 1)
        pk = _xor_shuffle(key, dist, lane)
        pi = _xor_shuffle(idx, dist, lane)
        beats = (key > pk) | ((key == pk) & (idx < pi))
        keep = beats == winner
        key = jnp.where(keep, key, pk)
        idx = jnp.where(keep, idx, pi)

    dn = (((1,), (0,)), ((), ()))
    vals = lax.dot_general(key, perm, dn, precision=lax.Precision.HIGHEST,
                           preferred_element_type=jnp.float32)
    idxf = lax.dot_general(idx.astype(jnp.float32), perm, dn,
                           precision=lax.Precision.HIGHEST,
                           preferred_element_type=jnp.float32)
    return vals, idxf.astype(jnp.int32)


def _topk_body(w_ref, p_ref, vals_ref, idx_ref):
    vals, idx = _topk_sort_block(w_ref[...], p_ref[...])
    vals_ref[...] = vals
    idx_ref[...] = idx


def _row_topk(embed_weight):
    v, h = embed_weight.shape
    lane = jnp.arange(h, dtype=jnp.int32)
    s = (lane & 127) * 32 + (lane >> 7)
    perm = (s[:, None] == jnp.arange(K, dtype=jnp.int32)[None, :]).astype(
        jnp.float32)
    return pl.pallas_call(
        _topk_body,
        grid=(v // ROWS_PER_BLOCK,),
        in_specs=[
            pl.BlockSpec((ROWS_PER_BLOCK, h), lambda i: (i, 0)),
            pl.BlockSpec((h, K), lambda i: (0, 0)),
        ],
        out_specs=[
            pl.BlockSpec((ROWS_PER_BLOCK, K), lambda i: (i, 0)),
            pl.BlockSpec((ROWS_PER_BLOCK, K), lambda i: (i, 0)),
        ],
        out_shape=[
            jax.ShapeDtypeStruct((v, K), jnp.float32),
            jax.ShapeDtypeStruct((v, K), jnp.int32),
        ],
    )(embed_weight, perm)


CHUNK = 128


@functools.cache
def _gather_fn(n_tokens):
    info = plsc.get_sparse_core_info()
    nc, ns = info.num_cores, info.num_subcores
    nw = nc * ns
    bpw = n_tokens // nw
    nchunks = bpw // CHUNK
    mesh = plsc.VectorSubcoreMesh(core_axis_name="c", subcore_axis_name="s")

    @functools.partial(
        pl.kernel,
        mesh=mesh,
        out_type=(
            jax.ShapeDtypeStruct((n_tokens, K), jnp.float32),
            jax.ShapeDtypeStruct((n_tokens, K), jnp.int32),
        ),
        scratch_types=[
            pltpu.VMEM((CHUNK,), jnp.int32),
            pltpu.VMEM((CHUNK, K), jnp.float32),
            pltpu.VMEM((CHUNK, K), jnp.int32),
            pltpu.SemaphoreType.DMA,
            pltpu.SemaphoreType.DMA,
        ],
    )
    def gather(tv_hbm, ti_hbm, ids_hbm, ov_hbm, oi_hbm, ids_v, vv, iv, s1, s2):
        wid = lax.axis_index("s") * nc + lax.axis_index("c")
        base = wid * bpw
        for c in range(nchunks):
            off = base + c * CHUNK
            pltpu.sync_copy(ids_hbm.at[pl.ds(off, CHUNK)], ids_v)
            cp1 = pltpu.async_copy(tv_hbm.at[ids_v], vv, s1)
            cp2 = pltpu.async_copy(ti_hbm.at[ids_v], iv, s2)
            cp1.wait()
            cp2.wait()
            pltpu.sync_copy(vv, ov_hbm.at[pl.ds(off, CHUNK)])
            pltpu.sync_copy(iv, oi_hbm.at[pl.ds(off, CHUNK)])

    return gather


def kernel(input_ids, embed_weight):
    b, s = input_ids.shape
    tvals, tidx = _row_topk(embed_weight)
    ov, oi = _gather_fn(b * s)(tvals, tidx, input_ids.reshape(-1))
    return ov.reshape(b, s, K), oi.reshape(b, s, K)


# speedup vs baseline: 6.4641x; 1.2882x over previous
import functools

import jax
import jax.numpy as jnp
from jax import lax
from jax.experimental import pallas as pl
from jax.experimental.pallas import tpu as pltpu
from jax.experimental.pallas import tpu_sc as plsc

K = 256
H = 4096
ROWS_PER_BLOCK = 8


HH = H // 2


def _half_stages():
    st = []
    kk = 2
    while kk <= K:
        j = kk // 2
        while j >= 1:
            st.append((j, kk))
            j //= 2
        kk *= 2
    for level in (1, 2):
        st.append((256 << (level - 1), None))
        j = 128
        while j >= 1:
            st.append((j, 256 << level))
            j //= 2
    st.append((1024, None))
    return st


_HSTAGES = _half_stages()


def _lane_dist(d):
    return 128 * d if d < 16 else d // 16


def _run_stage(key, idx, bit, d, dp):
    dist = _lane_dist(d)
    bitd = bit[dist]
    pk = jnp.where(bitd, jnp.roll(key, dist, axis=1),
                   jnp.roll(key, -dist, axis=1))
    pi = jnp.where(bitd, jnp.roll(idx, dist, axis=1),
                   jnp.roll(idx, -dist, axis=1))
    beats = (key > pk) | ((key == pk) & (idx < pi))
    if dp is None:
        sw = ~(beats ^ bitd)
    elif dp == "asc":
        sw = beats ^ bitd
    else:
        sw = beats ^ bitd ^ bit[_lane_dist(dp)]
    return jnp.where(sw, pk, key), jnp.where(sw, pi, idx)


def _topk_sort_block(x, perm):
    r = x.shape[0]
    lane = lax.broadcasted_iota(jnp.int32, (r, HH), 1)
    dists = sorted({_lane_dist(d) for d, _ in _HSTAGES}
                   | {_lane_dist(dp) for _, dp in _HSTAGES
                      if dp not in (None, "asc")}
                   | {_lane_dist(1 << b) for b in range(8)})
    bit = {dd: (lane & dd) != 0 for dd in dists}

    halves = []
    for hf in range(2):
        key = x[:, hf * HH:(hf + 1) * HH]
        idx = lane + hf * HH
        for d, dp in _HSTAGES:
            key, idx = _run_stage(key, idx, bit, d, dp)
        j = 128
        while j >= 1:
            key, idx = _run_stage(key, idx, bit, j,
                                  "asc" if hf == 0 else None)
            j //= 2
        halves.append((key, idx))

    (ka, ia), (kb, ib) = halves
    beats = (ka > kb) | ((ka == kb) & (ia < ib))
    key = jnp.where(beats, ka, kb)
    idx = jnp.where(beats, ia, ib)
    j = 128
    while j >= 1:
        key, idx = _run_stage(key, idx, bit, j, None)
        j //= 2

    dn = (((1,), (0,)), ((), ()))
    vals = lax.dot_general(key, perm, dn, precision=lax.Precision.HIGHEST,
                           preferred_element_type=jnp.float32)
    idxf = lax.dot_general(idx.astype(jnp.float32), perm, dn,
                           precision=lax.Precision.HIGHEST,
                           preferred_element_type=jnp.float32)
    return vals, idxf.astype(jnp.int32)


def _topk_body(w_ref, p_ref, vals_ref, idx_ref):
    vals, idx = _topk_sort_block(w_ref[...], p_ref[...])
    vals_ref[...] = vals
    idx_ref[...] = idx


def _row_topk(embed_weight):
    v, h = embed_weight.shape
    lane = jnp.arange(HH, dtype=jnp.int32)
    s = (lane & 127) * 16 + (lane >> 7)
    perm = ((s[:, None] == jnp.arange(K, dtype=jnp.int32)[None, :])
            & ((lane & 127) < 16)[:, None]).astype(jnp.float32)
    return pl.pallas_call(
        _topk_body,
        grid=(v // ROWS_PER_BLOCK,),
        in_specs=[
            pl.BlockSpec((ROWS_PER_BLOCK, h), lambda i: (i, 0)),
            pl.BlockSpec((HH, K), lambda i: (0, 0)),
        ],
        out_specs=[
            pl.BlockSpec((ROWS_PER_BLOCK, K), lambda i: (i, 0)),
            pl.BlockSpec((ROWS_PER_BLOCK, K), lambda i: (i, 0)),
        ],
        out_shape=[
            jax.ShapeDtypeStruct((v, K), jnp.float32),
            jax.ShapeDtypeStruct((v, K), jnp.int32),
        ],
    )(embed_weight, perm)


CHUNK = 128


@functools.cache
def _gather_fn(n_tokens):
    info = plsc.get_sparse_core_info()
    nc, ns = info.num_cores, info.num_subcores
    nw = nc * ns
    bpw = n_tokens // nw
    nchunks = bpw // CHUNK
    mesh = plsc.VectorSubcoreMesh(core_axis_name="c", subcore_axis_name="s")

    @functools.partial(
        pl.kernel,
        mesh=mesh,
        out_type=(
            jax.ShapeDtypeStruct((n_tokens, K), jnp.float32),
            jax.ShapeDtypeStruct((n_tokens, K), jnp.int32),
        ),
        scratch_types=[
            pltpu.VMEM((CHUNK,), jnp.int32),
            pltpu.VMEM((CHUNK, K), jnp.float32),
            pltpu.VMEM((CHUNK, K), jnp.int32),
            pltpu.SemaphoreType.DMA,
            pltpu.SemaphoreType.DMA,
        ],
    )
    def gather(tv_hbm, ti_hbm, ids_hbm, ov_hbm, oi_hbm, ids_v, vv, iv, s1, s2):
        wid = lax.axis_index("s") * nc + lax.axis_index("c")
        base = wid * bpw
        for c in range(nchunks):
            off = base + c * CHUNK
            pltpu.sync_copy(ids_hbm.at[pl.ds(off, CHUNK)], ids_v)
            cp1 = pltpu.async_copy(tv_hbm.at[ids_v], vv, s1)
            cp2 = pltpu.async_copy(ti_hbm.at[ids_v], iv, s2)
            cp1.wait()
            cp2.wait()
            pltpu.sync_copy(vv, ov_hbm.at[pl.ds(off, CHUNK)])
            pltpu.sync_copy(iv, oi_hbm.at[pl.ds(off, CHUNK)])

    return gather


def kernel(input_ids, embed_weight):
    b, s = input_ids.shape
    tvals, tidx = _row_topk(embed_weight)
    ov, oi = _gather_fn(b * s)(tvals, tidx, input_ids.reshape(-1))
    return ov.reshape(b, s, K), oi.reshape(b, s, K)


# speedup vs baseline: 8.1949x; 1.2677x over previous
import functools

import jax
import jax.numpy as jnp
from jax import lax
from jax.experimental import pallas as pl
from jax.experimental.pallas import tpu as pltpu
from jax.experimental.pallas import tpu_sc as plsc

K = 256
H = 4096
ROWS_PER_BLOCK = 16


HH = H // 2


def _half_stages():
    st = []
    kk = 2
    while kk <= K:
        j = kk // 2
        while j >= 1:
            st.append((j, kk))
            j //= 2
        kk *= 2
    for level in (1, 2):
        st.append((256 << (level - 1), None))
        j = 128
        while j >= 1:
            st.append((j, 256 << level))
            j //= 2
    st.append((1024, None))
    return st


_HSTAGES = _half_stages()


def _lane_dist(d):
    return 128 * d if d < 16 else d // 16


def _run_stage(key, idx, bit, d, dp):
    dist = _lane_dist(d)
    bitd = bit[dist]
    pk = jnp.where(bitd, jnp.roll(key, dist, axis=1),
                   jnp.roll(key, -dist, axis=1))
    pi = jnp.where(bitd, jnp.roll(idx, dist, axis=1),
                   jnp.roll(idx, -dist, axis=1))
    beats = (key > pk) | ((key == pk) & (idx < pi))
    if dp is None:
        sw = beats ^ bitd
        return jnp.where(sw, key, pk), jnp.where(sw, idx, pi)
    if dp == "asc":
        sw = beats ^ bitd
    else:
        sw = beats ^ bitd ^ bit[_lane_dist(dp)]
    return jnp.where(sw, pk, key), jnp.where(sw, pi, idx)


def _topk_sort_block(x, perm):
    r = x.shape[0]
    lane = lax.broadcasted_iota(jnp.int32, (r, HH), 1)
    dists = sorted({_lane_dist(d) for d, _ in _HSTAGES}
                   | {_lane_dist(dp) for _, dp in _HSTAGES
                      if dp not in (None, "asc")}
                   | {_lane_dist(1 << b) for b in range(8)})
    bit = {dd: (lane & dd) != 0 for dd in dists}

    ka, ia = x[:, :HH], lane
    kb, ib = x[:, HH:], lane + HH
    for d, dp in _HSTAGES:
        ka, ia = _run_stage(ka, ia, bit, d, dp)
        kb, ib = _run_stage(kb, ib, bit, d, dp)
    j = 128
    while j >= 1:
        ka, ia = _run_stage(ka, ia, bit, j, "asc")
        kb, ib = _run_stage(kb, ib, bit, j, None)
        j //= 2
    beats = (ka > kb) | ((ka == kb) & (ia < ib))
    key = jnp.where(beats, ka, kb)
    idx = jnp.where(beats, ia, ib)
    j = 128
    while j >= 1:
        key, idx = _run_stage(key, idx, bit, j, None)
        j //= 2

    dn = (((1,), (0,)), ((), ()))
    vals = lax.dot_general(key, perm, dn, precision=lax.Precision.HIGHEST,
                           preferred_element_type=jnp.float32)
    idxf = lax.dot_general(idx.astype(jnp.float32), perm, dn,
                           precision=lax.Precision.HIGHEST,
                           preferred_element_type=jnp.float32)
    return vals, idxf.astype(jnp.int32)


def _topk_body(w_ref, p_ref, vals_ref, idx_ref):
    vals, idx = _topk_sort_block(w_ref[...], p_ref[...])
    vals_ref[...] = vals
    idx_ref[...] = idx


def _row_topk(embed_weight):
    v, h = embed_weight.shape
    lane = jnp.arange(HH, dtype=jnp.int32)
    s = (lane & 127) * 16 + (lane >> 7)
    perm = ((s[:, None] == jnp.arange(K, dtype=jnp.int32)[None, :])
            & ((lane & 127) < 16)[:, None]).astype(jnp.float32)
    return pl.pallas_call(
        _topk_body,
        grid=(v // ROWS_PER_BLOCK,),
        in_specs=[
            pl.BlockSpec((ROWS_PER_BLOCK, h), lambda i: (i, 0)),
            pl.BlockSpec((HH, K), lambda i: (0, 0)),
        ],
        out_specs=[
            pl.BlockSpec((ROWS_PER_BLOCK, K), lambda i: (i, 0)),
            pl.BlockSpec((ROWS_PER_BLOCK, K), lambda i: (i, 0)),
        ],
        out_shape=[
            jax.ShapeDtypeStruct((v, K), jnp.float32),
            jax.ShapeDtypeStruct((v, K), jnp.int32),
        ],
    )(embed_weight, perm)


CHUNK = 128


@functools.cache
def _gather_fn(n_tokens):
    info = plsc.get_sparse_core_info()
    nc, ns = info.num_cores, info.num_subcores
    nw = nc * ns
    bpw = n_tokens // nw
    nchunks = bpw // CHUNK
    mesh = plsc.VectorSubcoreMesh(core_axis_name="c", subcore_axis_name="s")

    @functools.partial(
        pl.kernel,
        mesh=mesh,
        out_type=(
            jax.ShapeDtypeStruct((n_tokens, K), jnp.float32),
            jax.ShapeDtypeStruct((n_tokens, K), jnp.int32),
        ),
        scratch_types=[
            pltpu.VMEM((CHUNK,), jnp.int32),
            pltpu.VMEM((CHUNK, K), jnp.float32),
            pltpu.VMEM((CHUNK, K), jnp.int32),
            pltpu.SemaphoreType.DMA,
            pltpu.SemaphoreType.DMA,
        ],
    )
    def gather(tv_hbm, ti_hbm, ids_hbm, ov_hbm, oi_hbm, ids_v, vv, iv, s1, s2):
        wid = lax.axis_index("s") * nc + lax.axis_index("c")
        base = wid * bpw
        for c in range(nchunks):
            off = base + c * CHUNK
            pltpu.sync_copy(ids_hbm.at[pl.ds(off, CHUNK)], ids_v)
            cp1 = pltpu.async_copy(tv_hbm.at[ids_v], vv, s1)
            cp2 = pltpu.async_copy(ti_hbm.at[ids_v], iv, s2)
            cp1.wait()
            cp2.wait()
            pltpu.sync_copy(vv, ov_hbm.at[pl.ds(off, CHUNK)])
            pltpu.sync_copy(iv, oi_hbm.at[pl.ds(off, CHUNK)])

    return gather


def kernel(input_ids, embed_weight):
    b, s = input_ids.shape
    v = embed_weight.shape[0]
    vpad = -v % ROWS_PER_BLOCK
    if vpad:
        embed_weight = jnp.pad(embed_weight, ((0, vpad), (0, 0)))
    tvals, tidx = _row_topk(embed_weight)
    ov, oi = _gather_fn(b * s)(tvals, tidx, input_ids.reshape(-1))
    return ov.reshape(b, s, K), oi.reshape(b, s, K)
